# baseline (speedup 1.0000x reference)
import functools

import jax
import jax.numpy as jnp
from jax.experimental import pallas as pl
from jax.experimental.pallas import tpu as pltpu

K_NEIGHBORS = 50
DELTA = 1e-3
NEG_INF = float('-inf')
M_SLOTS = 5
N_SETS = 8
M2_SLOTS = 10
LANES = 128


def _enc_body(x_ref, w1_ref, b1_ref, w2_ref, b2_ref, key_ref, qa_ref):
    h = jax.lax.dot_general(
        x_ref[...], w1_ref[...], (((1,), (0,)), ((), ())),
        preferred_element_type=jnp.float32)
    h = jnp.maximum(h + b1_ref[...], 0.0)
    key = jax.lax.dot_general(
        h, w2_ref[...], (((1,), (0,)), ((), ())),
        preferred_element_type=jnp.float32) + b2_ref[...]
    key_ref[...] = key
    B = key.shape[0]
    qa_ref[...] = jnp.concatenate(
        [key, jnp.ones((B, 1), jnp.float32), jnp.zeros((B, 7), jnp.float32)],
        axis=1)


def _encoder(x, W1, b1, W2, b2):
    B = x.shape[0]
    D = W2.shape[1]
    return pl.pallas_call(
        _enc_body,
        out_shape=[
            jax.ShapeDtypeStruct((B, D), jnp.float32),
            jax.ShapeDtypeStruct((B, D + 8), jnp.float32),
        ],
    )(x, W1, b1.reshape(1, -1), W2, b2.reshape(1, -1))


def _aug_body(k_ref, out_ref, *, ct, C):
    c = pl.program_id(1)
    kk = k_ref[0]
    row = c * ct + jax.lax.broadcasted_iota(jnp.int32, (kk.shape[0], 1), 0)
    valid = row < C
    kk = jnp.where(valid, kk, 0.0)
    mk2h = jnp.sum(kk * kk, axis=1, keepdims=True) * 0.5
    tail = jnp.where(valid, -mk2h, -1e30)
    out_ref[0] = jnp.concatenate(
        [kk, tail, jnp.zeros((kk.shape[0], 7), jnp.float32)], axis=1)


def _augment_keys(mem_keys, ct):
    A, C, D = mem_keys.shape
    nc = pl.cdiv(C, ct)
    return pl.pallas_call(
        functools.partial(_aug_body, ct=ct, C=C),
        grid=(A, nc),
        in_specs=[pl.BlockSpec((1, ct, D), lambda a, c: (a, c, 0))],
        out_specs=pl.BlockSpec((1, ct, D + 8), lambda a, c: (a, c, 0)),
        out_shape=jax.ShapeDtypeStruct((A, nc * ct, D + 8), jnp.float32),
    )(mem_keys)


def _topk_body(q_ref, k_ref, vals_ref, idx_ref, kept_v, kept_i,
               cand_v_ref, cand_i_ref, *, ct, C, k, m):
    c = pl.program_id(2)
    nc = pl.num_programs(2)
    bt = q_ref.shape[0]

    @pl.when(c == 0)
    def _init():
        kept_v[...] = jnp.full_like(kept_v, NEG_INF)
        kept_i[...] = jnp.zeros_like(kept_i)

    neg = jax.lax.dot_general(
        q_ref[...], k_ref[0], (((1,), (1,)), ((), ())),
        preferred_element_type=jnp.float32)

    for h in range(ct // LANES):
        s = h % N_SETS
        x_v = neg[:, h * LANES:(h + 1) * LANES]
        x_i = (c * ct + h * LANES
               + jax.lax.broadcasted_iota(jnp.int32, (bt, LANES), 1))
        for j in range(m):
            sj = s * m + j
            kv = kept_v[sj]
            ki = kept_i[sj]
            take = x_v > kv
            kept_v[sj] = jnp.where(take, x_v, kv)
            kept_i[sj] = jnp.where(take, x_i, ki)
            x_v = jnp.where(take, kv, x_v)
            x_i = jnp.where(take, ki, x_i)

    @pl.when(c == nc - 1)
    def _emit():
        m2 = M2_SLOTS
        cand_v_ref[...] = jnp.full((bt, m2 * LANES), NEG_INF, jnp.float32)
        cand_i_ref[...] = jnp.zeros((bt, m2 * LANES), jnp.int32)
        def merge_slice(sj, _):
            x_v = kept_v[sj]
            x_i = kept_i[sj]
            for j in range(m2):
                sl = slice(j * LANES, (j + 1) * LANES)
                kv = cand_v_ref[:, sl]
                ki = cand_i_ref[:, sl]
                take = x_v > kv
                cand_v_ref[:, sl] = jnp.where(take, x_v, kv)
                cand_i_ref[:, sl] = jnp.where(take, x_i, ki)
                x_v = jnp.where(take, kv, x_v)
                x_i = jnp.where(take, ki, x_i)
            return 0

        jax.lax.fori_loop(0, N_SETS * m, merge_slice, 0)
        W = m2 * LANES
        pos = jax.lax.broadcasted_iota(jnp.int32, (bt, W), 1)
        kw = vals_ref.shape[2]
        slot = jax.lax.broadcasted_iota(jnp.int32, (bt, kw), 1)

        def body(t, _):
            cv = cand_v_ref[...]
            cur = jnp.max(cv, axis=1, keepdims=True)
            am = jnp.argmax(cv, axis=1)
            hot = pos == am[:, None]
            idx = jnp.sum(jnp.where(hot, cand_i_ref[...], 0),
                          axis=1, keepdims=True)
            cand_v_ref[...] = jnp.where(hot, NEG_INF, cv)
            sel = slot == t
            vals_ref[0] = jnp.where(sel, cur, vals_ref[0])
            idx_ref[0] = jnp.where(sel, idx, idx_ref[0])
            return 0

        jax.lax.fori_loop(0, k, body, 0)


def _fused_topk(qa, aug_keys, A, C, bt=256, ct=2048, k=K_NEIGHBORS,
                m=M_SLOTS):
    B, Da = qa.shape
    nc = aug_keys.shape[1] // ct
    grid = (A, B // bt, nc)
    kw = 64
    body = functools.partial(_topk_body, ct=ct, C=C, k=k, m=m)
    vals, idx = pl.pallas_call(
        body,
        grid=grid,
        in_specs=[
            pl.BlockSpec((bt, Da), lambda a, b, c: (b, 0)),
            pl.BlockSpec((1, ct, Da), lambda a, b, c: (a, c, 0)),
        ],
        out_specs=[
            pl.BlockSpec((1, bt, kw), lambda a, b, c: (a, b, 0)),
            pl.BlockSpec((1, bt, kw), lambda a, b, c: (a, b, 0)),
        ],
        out_shape=[
            jax.ShapeDtypeStruct((A, B, kw), jnp.float32),
            jax.ShapeDtypeStruct((A, B, kw), jnp.int32),
        ],
        scratch_shapes=[
            pltpu.VMEM((N_SETS * m, bt, LANES), jnp.float32),
            pltpu.VMEM((N_SETS * m, bt, LANES), jnp.int32),
            pltpu.VMEM((bt, M2_SLOTS * LANES), jnp.float32),
            pltpu.VMEM((bt, M2_SLOTS * LANES), jnp.int32),
        ],
        compiler_params=pltpu.CompilerParams(
            dimension_semantics=("parallel", "parallel", "arbitrary")),
    )(qa, aug_keys)
    return vals[:, :, :k], idx[:, :, :k]


def kernel(x, W1, b1, W2, b2, mem_keys, mem_values):
    A, C, _ = mem_keys.shape
    ct = 2048
    key, qa = _encoder(x, W1, b1, W2, b2)
    aug_keys = _augment_keys(mem_keys, ct)
    top_half, top_idx = _fused_topk(qa, aug_keys, A, C, ct=ct)
    q2 = jnp.sum(key * key, axis=1)
    dists = q2[None, :, None] - 2.0 * top_half
    w = 1.0 / (dists + DELTA)
    w = w / jnp.sum(w, axis=-1, keepdims=True)
    v = jax.vmap(lambda mv, ti: mv[ti])(mem_values, top_idx)
    q_vals = jnp.sum(w * v, axis=-1)
    values = q_vals.T
    action = jnp.argmax(values, axis=1)
    indexes = jnp.transpose(top_idx, (1, 0, 2))
    scores = jnp.transpose(w, (1, 0, 2))
    return (key, values, action, indexes, scores)


# speedup vs baseline: 1.0703x; 1.0703x over previous
import functools

import jax
import jax.numpy as jnp
from jax.experimental import pallas as pl
from jax.experimental.pallas import tpu as pltpu

K_NEIGHBORS = 50
DELTA = 1e-3
NEG_INF = float('-inf')
M_SLOTS = 5
N_SETS = 8
M2_SLOTS = 10
LANES = 128


def _enc_body(x_ref, w1_ref, b1_ref, w2_ref, b2_ref, key_ref):
    h = jax.lax.dot_general(
        x_ref[...], w1_ref[...], (((1,), (0,)), ((), ())),
        preferred_element_type=jnp.float32)
    h = jnp.maximum(h + b1_ref[...], 0.0)
    key_ref[...] = jax.lax.dot_general(
        h, w2_ref[...], (((1,), (0,)), ((), ())),
        preferred_element_type=jnp.float32) + b2_ref[...]


def _encoder(x, W1, b1, W2, b2):
    B = x.shape[0]
    D = W2.shape[1]
    return pl.pallas_call(
        _enc_body,
        out_shape=jax.ShapeDtypeStruct((B, D), jnp.float32),
    )(x, W1, b1.reshape(1, -1), W2, b2.reshape(1, -1))


def _topk_body(q_ref, k_ref, vals_ref, idx_ref, kept_v, kept_i,
               cand_v_ref, cand_i_ref, *, ct, C, k, m):
    c = pl.program_id(2)
    nc = pl.num_programs(2)
    bt = q_ref.shape[0]

    @pl.when(c == 0)
    def _init():
        kept_v[...] = jnp.full_like(kept_v, NEG_INF)
        kept_i[...] = jnp.zeros_like(kept_i)

    q = q_ref[...]
    kk = k_ref[0]
    qk = jax.lax.dot_general(
        q, kk, (((1,), (1,)), ((), ())), preferred_element_type=jnp.float32)
    q2 = jnp.sum(q * q, axis=1, keepdims=True)
    mk2 = jnp.sum(kk * kk, axis=1)[None, :]
    neg = (2.0 * qk - q2) - mk2

    col = c * ct + jax.lax.broadcasted_iota(jnp.int32, neg.shape, 1)
    neg = jnp.where(col < C, neg, NEG_INF)

    for h in range(ct // LANES):
        s = h % N_SETS
        x_v = neg[:, h * LANES:(h + 1) * LANES]
        x_i = (c * ct + h * LANES
               + jax.lax.broadcasted_iota(jnp.int32, (bt, LANES), 1))
        for j in range(m):
            sj = s * m + j
            kv = kept_v[sj]
            ki = kept_i[sj]
            take = x_v > kv
            kept_v[sj] = jnp.where(take, x_v, kv)
            kept_i[sj] = jnp.where(take, x_i, ki)
            x_v = jnp.where(take, kv, x_v)
            x_i = jnp.where(take, ki, x_i)

    @pl.when(c == nc - 1)
    def _emit():
        m2 = M2_SLOTS
        cand_v_ref[...] = jnp.full((bt, m2 * LANES), NEG_INF, jnp.float32)
        cand_i_ref[...] = jnp.zeros((bt, m2 * LANES), jnp.int32)

        def merge_slice(sj, _):
            x_v = kept_v[sj]
            x_i = kept_i[sj]
            for j in range(m2):
                sl = slice(j * LANES, (j + 1) * LANES)
                kv = cand_v_ref[:, sl]
                ki = cand_i_ref[:, sl]
                take = x_v > kv
                cand_v_ref[:, sl] = jnp.where(take, x_v, kv)
                cand_i_ref[:, sl] = jnp.where(take, x_i, ki)
                x_v = jnp.where(take, kv, x_v)
                x_i = jnp.where(take, ki, x_i)
            return 0

        jax.lax.fori_loop(0, N_SETS * m, merge_slice, 0)
        W = m2 * LANES
        pos = jax.lax.broadcasted_iota(jnp.int32, (bt, W), 1)
        kw = vals_ref.shape[2]
        slot = jax.lax.broadcasted_iota(jnp.int32, (bt, kw), 1)

        def body(t, _):
            cv = cand_v_ref[...]
            cur = jnp.max(cv, axis=1, keepdims=True)
            am = jnp.argmax(cv, axis=1)
            hot = pos == am[:, None]
            idx = jnp.sum(jnp.where(hot, cand_i_ref[...], 0),
                          axis=1, keepdims=True)
            cand_v_ref[...] = jnp.where(hot, NEG_INF, cv)
            sel = slot == t
            vals_ref[0] = jnp.where(sel, cur, vals_ref[0])
            idx_ref[0] = jnp.where(sel, idx, idx_ref[0])
            return 0

        jax.lax.fori_loop(0, k, body, 0)


def _fused_topk(q, mem_keys, bt=256, ct=2048, k=K_NEIGHBORS, m=M_SLOTS):
    A, C, D = mem_keys.shape
    B = q.shape[0]
    nc = pl.cdiv(C, ct)
    grid = (A, B // bt, nc)
    kw = 64
    body = functools.partial(_topk_body, ct=ct, C=C, k=k, m=m)
    vals, idx = pl.pallas_call(
        body,
        grid=grid,
        in_specs=[
            pl.BlockSpec((bt, D), lambda a, b, c: (b, 0)),
            pl.BlockSpec((1, ct, D), lambda a, b, c: (a, c, 0)),
        ],
        out_specs=[
            pl.BlockSpec((1, bt, kw), lambda a, b, c: (a, b, 0)),
            pl.BlockSpec((1, bt, kw), lambda a, b, c: (a, b, 0)),
        ],
        out_shape=[
            jax.ShapeDtypeStruct((A, B, kw), jnp.float32),
            jax.ShapeDtypeStruct((A, B, kw), jnp.int32),
        ],
        scratch_shapes=[
            pltpu.VMEM((N_SETS * m, bt, LANES), jnp.float32),
            pltpu.VMEM((N_SETS * m, bt, LANES), jnp.int32),
            pltpu.VMEM((bt, M2_SLOTS * LANES), jnp.float32),
            pltpu.VMEM((bt, M2_SLOTS * LANES), jnp.int32),
        ],
        compiler_params=pltpu.CompilerParams(
            dimension_semantics=("parallel", "parallel", "arbitrary")),
    )(q, mem_keys)
    return vals[:, :, :k], idx[:, :, :k]


def kernel(x, W1, b1, W2, b2, mem_keys, mem_values):
    key = _encoder(x, W1, b1, W2, b2)
    top_neg, top_idx = _fused_topk(key, mem_keys)
    dists = -top_neg
    w = 1.0 / (dists + DELTA)
    w = w / jnp.sum(w, axis=-1, keepdims=True)
    v = jax.vmap(lambda mv, ti: mv[ti])(mem_values, top_idx)
    q_vals = jnp.sum(w * v, axis=-1)
    values = q_vals.T
    action = jnp.argmax(values, axis=1)
    indexes = jnp.transpose(top_idx, (1, 0, 2))
    scores = jnp.transpose(w, (1, 0, 2))
    return (key, values, action, indexes, scores)


# speedup vs baseline: 1.1104x; 1.0375x over previous
import functools

import jax
import jax.numpy as jnp
from jax.experimental import pallas as pl
from jax.experimental.pallas import tpu as pltpu

K_NEIGHBORS = 50
DELTA = 1e-3
NEG_INF = float('-inf')
M_SLOTS = 5
N_SETS = 8
M2_SLOTS = 10
LANES = 128


def _enc_body(x_ref, w1_ref, b1_ref, w2_ref, b2_ref, key_ref):
    h = jax.lax.dot_general(
        x_ref[...], w1_ref[...], (((1,), (0,)), ((), ())),
        preferred_element_type=jnp.float32)
    h = jnp.maximum(h + b1_ref[...], 0.0)
    key_ref[...] = jax.lax.dot_general(
        h, w2_ref[...], (((1,), (0,)), ((), ())),
        preferred_element_type=jnp.float32) + b2_ref[...]


def _encoder(x, W1, b1, W2, b2):
    B = x.shape[0]
    D = W2.shape[1]
    return pl.pallas_call(
        _enc_body,
        out_shape=jax.ShapeDtypeStruct((B, D), jnp.float32),
    )(x, W1, b1.reshape(1, -1), W2, b2.reshape(1, -1))


def _topk_body(q_ref, k_ref, vals_ref, idx_ref, kept_v, kept_i,
               cand_v_ref, cand_i_ref, *, ct, C, k, m):
    c = pl.program_id(2)
    nc = pl.num_programs(2)
    bt = q_ref.shape[0]

    @pl.when(c == 0)
    def _init():
        kept_v[...] = jnp.full_like(kept_v, NEG_INF)
        kept_i[...] = jnp.zeros_like(kept_i)

    q = q_ref[...]
    kk = k_ref[0]
    qk = jax.lax.dot_general(
        q, kk, (((1,), (1,)), ((), ())), preferred_element_type=jnp.float32)
    q2 = jnp.sum(q * q, axis=1, keepdims=True)
    mk2 = jnp.sum(kk * kk, axis=1)[None, :]
    neg = (2.0 * qk - q2) - mk2

    col = c * ct + jax.lax.broadcasted_iota(jnp.int32, neg.shape, 1)
    neg = jnp.where(col < C, neg, NEG_INF)

    for h in range(ct // LANES):
        s = h % N_SETS
        x_v = neg[:, h * LANES:(h + 1) * LANES]
        x_i = (c * ct + h * LANES
               + jax.lax.broadcasted_iota(jnp.int32, (bt, LANES), 1))
        for j in range(m):
            sj = s * m + j
            kv = kept_v[sj]
            ki = kept_i[sj]
            take = x_v > kv
            kept_v[sj] = jnp.where(take, x_v, kv)
            kept_i[sj] = jnp.where(take, x_i, ki)
            x_v = jnp.where(take, kv, x_v)
            x_i = jnp.where(take, ki, x_i)

    @pl.when(c == nc - 1)
    def _emit():
        m2 = M2_SLOTS
        cand_v_ref[...] = jnp.full((bt, m2 * LANES), NEG_INF, jnp.float32)
        cand_i_ref[...] = jnp.zeros((bt, m2 * LANES), jnp.int32)

        def merge_slice(sj, _):
            x_v = kept_v[sj]
            x_i = kept_i[sj]
            for j in range(m2):
                sl = slice(j * LANES, (j + 1) * LANES)
                kv = cand_v_ref[:, sl]
                ki = cand_i_ref[:, sl]
                take = x_v > kv
                cand_v_ref[:, sl] = jnp.where(take, x_v, kv)
                cand_i_ref[:, sl] = jnp.where(take, x_i, ki)
                x_v = jnp.where(take, kv, x_v)
                x_i = jnp.where(take, ki, x_i)
            return 0

        jax.lax.fori_loop(0, N_SETS * m, merge_slice, 0)
        W = m2 * LANES
        pos = jax.lax.broadcasted_iota(jnp.int32, (bt, W), 1)
        kw = vals_ref.shape[2]
        slot = jax.lax.broadcasted_iota(jnp.int32, (bt, kw), 1)

        def body(t, _):
            cv = cand_v_ref[...]
            cur = jnp.max(cv, axis=1, keepdims=True)
            am = jnp.argmax(cv, axis=1)
            hot = pos == am[:, None]
            idx = jnp.sum(jnp.where(hot, cand_i_ref[...], 0),
                          axis=1, keepdims=True)
            cand_v_ref[...] = jnp.where(hot, NEG_INF, cv)
            sel = slot == t
            vals_ref[0] = jnp.where(sel, cur, vals_ref[0])
            idx_ref[0] = jnp.where(sel, idx, idx_ref[0])
            return 0

        jax.lax.fori_loop(0, k, body, 0)


def _fused_topk(q, mem_keys, bt=512, ct=2048, k=K_NEIGHBORS, m=M_SLOTS):
    A, C, D = mem_keys.shape
    B = q.shape[0]
    nc = pl.cdiv(C, ct)
    grid = (A, B // bt, nc)
    kw = 64
    body = functools.partial(_topk_body, ct=ct, C=C, k=k, m=m)
    vals, idx = pl.pallas_call(
        body,
        grid=grid,
        in_specs=[
            pl.BlockSpec((bt, D), lambda a, b, c: (b, 0)),
            pl.BlockSpec((1, ct, D), lambda a, b, c: (a, c, 0)),
        ],
        out_specs=[
            pl.BlockSpec((1, bt, kw), lambda a, b, c: (a, b, 0)),
            pl.BlockSpec((1, bt, kw), lambda a, b, c: (a, b, 0)),
        ],
        out_shape=[
            jax.ShapeDtypeStruct((A, B, kw), jnp.float32),
            jax.ShapeDtypeStruct((A, B, kw), jnp.int32),
        ],
        scratch_shapes=[
            pltpu.VMEM((N_SETS * m, bt, LANES), jnp.float32),
            pltpu.VMEM((N_SETS * m, bt, LANES), jnp.int32),
            pltpu.VMEM((bt, M2_SLOTS * LANES), jnp.float32),
            pltpu.VMEM((bt, M2_SLOTS * LANES), jnp.int32),
        ],
        compiler_params=pltpu.CompilerParams(
            dimension_semantics=("parallel", "parallel", "arbitrary")),
    )(q, mem_keys)
    return vals[:, :, :k], idx[:, :, :k]


def kernel(x, W1, b1, W2, b2, mem_keys, mem_values):
    key = _encoder(x, W1, b1, W2, b2)
    top_neg, top_idx = _fused_topk(key, mem_keys)
    dists = -top_neg
    w = 1.0 / (dists + DELTA)
    w = w / jnp.sum(w, axis=-1, keepdims=True)
    v = jax.vmap(lambda mv, ti: mv[ti])(mem_values, top_idx)
    q_vals = jnp.sum(w * v, axis=-1)
    values = q_vals.T
    action = jnp.argmax(values, axis=1)
    indexes = jnp.transpose(top_idx, (1, 0, 2))
    scores = jnp.transpose(w, (1, 0, 2))
    return (key, values, action, indexes, scores)
